# Initial kernel scaffold; baseline (speedup 1.0000x reference)
import jax
import jax.numpy as jnp
from jax.experimental import pallas as pl


def kernel(x, emb_loc):
    raise NotImplementedError("write your pallas kernel here")


# speedup vs baseline: 1.7977x; 1.7977x over previous
import functools

import jax
import jax.numpy as jnp
from jax import lax
from jax.experimental import pallas as pl
from jax.experimental.pallas import tpu as pltpu
from jax.experimental.pallas import tpu_sc as plsc

_IDXW = 128
_CH_IDXROWS = 4


@functools.lru_cache(maxsize=None)
def _make_gather(n_rows: int, d: int, v: int):
    info = plsc.get_sparse_core_info()
    nw = info.num_cores * info.num_subcores
    assert n_rows % (nw * _IDXW * _CH_IDXROWS) == 0
    idxrows_per_w = n_rows // (nw * _IDXW)
    n_chunks = idxrows_per_w // _CH_IDXROWS
    ch_rows = _CH_IDXROWS * _IDXW

    mesh = plsc.VectorSubcoreMesh(core_axis_name="c", subcore_axis_name="s")

    @functools.partial(
        pl.kernel,
        mesh=mesh,
        out_type=jax.ShapeDtypeStruct((n_rows, d), jnp.float32),
        scratch_types=[
            pltpu.VMEM((_CH_IDXROWS, _IDXW), jnp.int32),
            pltpu.VMEM((ch_rows, d), jnp.float32),
            pltpu.SemaphoreType.DMA,
        ],
        compiler_params=pltpu.CompilerParams(use_tc_tiling_on_sc=False),
    )
    def gather(table_hbm, idx_hbm, out_hbm, idx_v, rows_v, sem):
        wid = lax.axis_index("s") * info.num_cores + lax.axis_index("c")
        base_idxrow = wid * idxrows_per_w

        def body(g, carry):
            r0 = base_idxrow + g * _CH_IDXROWS
            pltpu.sync_copy(idx_hbm.at[pl.ds(r0, _CH_IDXROWS)], idx_v)
            copies = [
                pltpu.async_copy(
                    table_hbm.at[idx_v.at[j]],
                    rows_v.at[pl.ds(j * _IDXW, _IDXW)],
                    sem,
                )
                for j in range(_CH_IDXROWS)
            ]
            for c in copies:
                c.wait()
            pltpu.sync_copy(rows_v, out_hbm.at[pl.ds(r0 * _IDXW, ch_rows)])
            return carry

        lax.fori_loop(0, n_chunks, body, 0, unroll=False)

    return gather


def kernel(x, emb_loc):
    b, h = x.shape
    v, d = emb_loc.shape
    n = b * h
    idx2d = x.reshape(n // _IDXW, _IDXW)
    out = _make_gather(n, d, v)(emb_loc, idx2d)
    return out.reshape(b, h, d)


# speedup vs baseline: 1.8740x; 1.0425x over previous
import functools

import jax
import jax.numpy as jnp
from jax import lax
from jax.experimental import pallas as pl
from jax.experimental.pallas import tpu as pltpu
from jax.experimental.pallas import tpu_sc as plsc

_IDXW = 128
_CH_IDXROWS = 4


@functools.lru_cache(maxsize=None)
def _make_gather(n_rows: int, d: int, v: int):
    info = plsc.get_sparse_core_info()
    nw = info.num_cores * info.num_subcores
    assert n_rows % (nw * _IDXW * _CH_IDXROWS * 2) == 0
    idxrows_per_w = n_rows // (nw * _IDXW)
    n_chunks = idxrows_per_w // _CH_IDXROWS
    ch_rows = _CH_IDXROWS * _IDXW

    mesh = plsc.VectorSubcoreMesh(core_axis_name="c", subcore_axis_name="s")

    @functools.partial(
        pl.kernel,
        mesh=mesh,
        out_type=jax.ShapeDtypeStruct((n_rows, d), jnp.float32),
        scratch_types=[
            pltpu.VMEM((idxrows_per_w, _IDXW), jnp.int32),
            pltpu.VMEM((2, ch_rows, d), jnp.float32),
            pltpu.SemaphoreType.DMA,
            pltpu.SemaphoreType.DMA,
        ],
        compiler_params=pltpu.CompilerParams(use_tc_tiling_on_sc=False),
    )
    def gather(table_hbm, idx_hbm, out_hbm, idx_v, rows_v, sem_g, sem_o):
        wid = lax.axis_index("s") * info.num_cores + lax.axis_index("c")
        base_idxrow = wid * idxrows_per_w
        base_row = base_idxrow * _IDXW

        pltpu.sync_copy(idx_hbm.at[pl.ds(base_idxrow, idxrows_per_w)], idx_v)

        def fire(k, b):
            for j in range(_CH_IDXROWS):
                pltpu.async_copy(
                    table_hbm.at[idx_v.at[k * _CH_IDXROWS + j]],
                    rows_v.at[b, pl.ds(j * _IDXW, _IDXW)],
                    sem_g,
                )

        def wait_gathers(b):
            pltpu.make_async_copy(
                table_hbm.at[pl.ds(0, ch_rows)], rows_v.at[b], sem_g
            ).wait()

        def writeback(k, b):
            pltpu.async_copy(
                rows_v.at[b], out_hbm.at[pl.ds(base_row + k * ch_rows, ch_rows)],
                sem_o,
            )

        def drain_writeback():
            pltpu.make_async_copy(
                rows_v.at[0], out_hbm.at[pl.ds(base_row, ch_rows)], sem_o
            ).wait()

        fire(0, 0)

        def body(g, carry):
            k0 = g * 2
            @pl.when(g > 0)
            def _():
                drain_writeback()
            fire(k0 + 1, 1)
            wait_gathers(0)
            writeback(k0, 0)
            drain_writeback()

            @pl.when(g < n_chunks // 2 - 1)
            def _():
                fire(k0 + 2, 0)
            wait_gathers(1)
            writeback(k0 + 1, 1)
            return carry

        lax.fori_loop(0, n_chunks // 2, body, 0, unroll=False)
        drain_writeback()

    return gather


def kernel(x, emb_loc):
    b, h = x.shape
    v, d = emb_loc.shape
    n = b * h
    idx2d = x.reshape(n // _IDXW, _IDXW)
    out = _make_gather(n, d, v)(emb_loc, idx2d)
    return out.reshape(b, h, d)
